# baseline (speedup 1.0000x reference)
import jax
import jax.numpy as jnp
from jax.experimental import pallas as pl

_K = 384
_SIGN = -(2 ** 31)
_LOW31 = 0x7FFFFFFF


def _sim_body(x_ref, y_ref, sim_ref):
    x = x_ref[...]
    y = y_ref[...]
    dot = jnp.sum(x * y, axis=-1)
    nx = jnp.maximum(jnp.sqrt(jnp.sum(x * x, axis=-1)), 1e-8)
    ny = jnp.maximum(jnp.sqrt(jnp.sum(y * y, axis=-1)), 1e-8)
    sim_ref[...] = (dot / (nx * ny))[:, None, :]


def _weights_body(sim_ref, w_ref):
    s = sim_ref[:, 0, :]
    B, C = s.shape
    bits = jax.lax.bitcast_convert_type(s, jnp.int32)
    m = jnp.where(bits >= 0, bits, bits ^ _LOW31)

    def bit_step(i, t):
        bit = jax.lax.shift_left(jnp.int32(1), jnp.int32(31) - i)
        cand = t | bit
        cnt = jnp.sum(jnp.where(m < (cand ^ _SIGN), 1, 0), axis=1,
                      keepdims=True)
        return jnp.where(cnt <= _K - 1, cand, t)

    t = jax.lax.fori_loop(0, 32, bit_step, jnp.zeros((B, 1), jnp.int32))
    tm = t ^ _SIGN
    less = m < tm
    eq = m == tm
    cnt_less = jnp.sum(jnp.where(less, 1, 0), axis=1, keepdims=True)
    need = _K - cnt_less

    col = jax.lax.broadcasted_iota(jnp.int32, (B, C), 1)

    def cut_step(_, lohi):
        lo, hi = lohi
        mid = (lo + hi) // 2
        c = jnp.sum(jnp.where(eq & (col < mid), 1, 0), axis=1,
                    keepdims=True)
        ok = c >= need
        return jnp.where(ok, lo, mid), jnp.where(ok, mid, hi)

    lo0 = jnp.zeros((B, 1), jnp.int32)
    hi0 = jnp.full((B, 1), C, jnp.int32)
    _, hi = jax.lax.fori_loop(0, 10, cut_step, (lo0, hi0))
    sel = less | (eq & (col < hi))

    t_f = jax.lax.bitcast_convert_type(
        jnp.where(tm >= 0, tm, tm ^ _LOW31), jnp.float32)
    e = jnp.where(sel, jnp.exp(s - t_f), 0.0)
    w_ref[:, 0, :] = e / jnp.sum(e, axis=1, keepdims=True)


def _scale_body(w_ref, x_ref, o_ref):
    o_ref[...] = x_ref[...] * w_ref[...][:, 0, :, None]


def kernel(x, y):
    B, C, H, W = x.shape
    S = H * W
    xr = x.reshape(B, C, S)
    yr = y.reshape(B, C, S)

    sim = pl.pallas_call(
        _sim_body,
        grid=(B,),
        in_specs=[pl.BlockSpec((1, C, S), lambda b: (b, 0, 0)),
                  pl.BlockSpec((1, C, S), lambda b: (b, 0, 0))],
        out_specs=pl.BlockSpec((1, 1, C), lambda b: (b, 0, 0)),
        out_shape=jax.ShapeDtypeStruct((B, 1, C), jnp.float32),
    )(xr, yr)

    w = pl.pallas_call(
        _weights_body,
        out_shape=jax.ShapeDtypeStruct((B, 1, C), jnp.float32),
    )(sim)

    out = pl.pallas_call(
        _scale_body,
        grid=(B,),
        in_specs=[pl.BlockSpec((1, 1, C), lambda b: (b, 0, 0)),
                  pl.BlockSpec((1, C, S), lambda b: (b, 0, 0))],
        out_specs=pl.BlockSpec((1, C, S), lambda b: (b, 0, 0)),
        out_shape=jax.ShapeDtypeStruct((B, C, S), jnp.float32),
    )(w, xr)
    return out.reshape(B, C, H, W)
